# Initial kernel scaffold; baseline (speedup 1.0000x reference)
import functools

import jax
import jax.numpy as jnp
from jax import lax
from jax.experimental import pallas as pl
from jax.experimental.pallas import tpu as pltpu
from jax.experimental.pallas import tpu_sc as plsc

VOCAB = 100000
DIM = 128
HID = 1024
N_DENO = 41
N_CONO = 2
B = 16384
L = 50

NC = 2
NS = 16
NW = NC * NS

SAMP_PER_W = B // NW
CHUNK_SAMP = 8
ROWS_PER_CHUNK = CHUNK_SAMP * L
SUB = 4
SUB_ROWS = ROWS_PER_CHUNK // SUB
N_CHUNK = SAMP_PER_W // CHUNK_SAMP

_sc_mesh = plsc.VectorSubcoreMesh(core_axis_name="c", subcore_axis_name="s")


@functools.partial(
    pl.kernel,
    mesh=_sc_mesh,
    out_type=[
        jax.ShapeDtypeStruct((B * L, DIM), jnp.float32),
        jax.ShapeDtypeStruct((B, DIM), jnp.float32),
    ],
    scratch_types=[
        pltpu.VMEM((SUB, SUB_ROWS), jnp.int32),
        pltpu.VMEM((ROWS_PER_CHUNK, DIM), jnp.float32),
        pltpu.VMEM((CHUNK_SAMP, DIM), jnp.float32),
        pltpu.SemaphoreType.DMA,
    ],
)
def _gather_mean(ids_hbm, table_hbm, vecs_hbm, repr_hbm, idx_v, rows_v, repr_v, sem):
    wid = lax.axis_index("s") * NC + lax.axis_index("c")
    base_samp = wid * SAMP_PER_W

    def chunk_body(ci, carry):
        samp0 = base_samp + ci * CHUNK_SAMP
        row0 = samp0 * L
        pltpu.sync_copy(ids_hbm.at[pl.ds(row0 // SUB_ROWS, SUB), :], idx_v)
        descs = []
        for k in range(SUB):
            descs.append(
                pltpu.async_copy(
                    table_hbm.at[idx_v.at[k]],
                    rows_v.at[pl.ds(k * SUB_ROWS, SUB_ROWS), :],
                    sem,
                )
            )
        for d in descs:
            d.wait()
        for j in range(CHUNK_SAMP):
            def rbody(r, acc):
                return tuple(
                    acc[c] + rows_v[j * L + r, pl.ds(c * 16, 16)] for c in range(8)
                )
            acc0 = tuple(jnp.zeros((16,), jnp.float32) for _ in range(8))
            acc = lax.fori_loop(0, L, rbody, acc0)
            for c in range(8):
                repr_v[j, pl.ds(c * 16, 16)] = acc[c] * (1.0 / L)
        pltpu.sync_copy(rows_v, vecs_hbm.at[pl.ds(row0, ROWS_PER_CHUNK), :])
        pltpu.sync_copy(repr_v, repr_hbm.at[pl.ds(samp0, CHUNK_SAMP), :])
        return carry

    lax.fori_loop(0, N_CHUNK, chunk_body, 0)


BM = 1024
NB = B // BM
PAD = 128


def _probe_block(x, W1, b1, W2, b2, W3, b3):
    h = jnp.maximum(jnp.dot(x, W1, preferred_element_type=jnp.float32) + b1, 0.0)
    h = jnp.maximum(jnp.dot(h, W2, preferred_element_type=jnp.float32) + b2, 0.0)
    return jnp.dot(h, W3, preferred_element_type=jnp.float32) + b3


def _log_softmax(logits):
    m = jnp.max(logits, axis=1, keepdims=True)
    s = logits - m
    lse = jnp.log(jnp.sum(jnp.exp(s), axis=1, keepdims=True))
    return s - lse


def _losses_kernel(
    repr_ref, dW1, db1, dW2, db2, dW3, db3, cW1, cb1, cW2, cb2, cW3, cb3,
    dlab_ref, clab_ref, deno_sum, cono_sum, adv_sum,
):
    i = pl.program_id(0)

    @pl.when(i == 0)
    def _():
        deno_sum[0, 0] = 0.0
        cono_sum[0, 0] = 0.0
        adv_sum[0, 0] = 0.0

    x = repr_ref[...]

    cols = lax.broadcasted_iota(jnp.int32, (BM, PAD), 1)

    dlogits = _probe_block(x, dW1[...], db1[...], dW2[...], db2[...], dW3[...], db3[...])
    dlogp = _log_softmax(dlogits)
    d_onehot = (cols == dlab_ref[0]).astype(jnp.float32)
    deno_sum[0, 0] += jnp.sum(dlogp * d_onehot)

    clogits = _probe_block(x, cW1[...], cb1[...], cW2[...], cb2[...], cW3[...], cb3[...])
    clogp = _log_softmax(clogits)
    c_onehot = (cols == clab_ref[0]).astype(jnp.float32)
    cono_sum[0, 0] += jnp.sum(clogp * c_onehot)

    u = 1.0 / N_CONO
    kl_terms = jnp.where(cols < N_CONO, u * (jnp.log(u) - clogp), 0.0)
    adv_sum[0, 0] += jnp.sum(kl_terms)


def _run_losses(seq_repr, dW1, db1, dW2, db2, dW3, db3, cW1, cb1, cW2, cb2, cW3, cb3,
                deno_labels, cono_labels):
    dW3p = jnp.pad(dW3, ((0, 0), (0, PAD - N_DENO)))
    db3p = jnp.pad(db3.reshape(1, N_DENO), ((0, 0), (0, PAD - N_DENO)),
                   constant_values=-1e30)
    cW3p = jnp.pad(cW3, ((0, 0), (0, PAD - N_CONO)))
    cb3p = jnp.pad(cb3.reshape(1, N_CONO), ((0, 0), (0, PAD - N_CONO)),
                   constant_values=-1e30)

    dlab = deno_labels.astype(jnp.int32).reshape(NB, BM, 1)
    clab = cono_labels.astype(jnp.int32).reshape(NB, BM, 1)

    full = lambda shape: pl.BlockSpec(shape, lambda i: (0,) * len(shape))
    grid_spec = pl.GridSpec(
        grid=(NB,),
        in_specs=[
            pl.BlockSpec((BM, DIM), lambda i: (i, 0)),
            full((DIM, HID)), full((1, HID)),
            full((HID, HID)), full((1, HID)),
            full((HID, PAD)), full((1, PAD)),
            full((DIM, HID)), full((1, HID)),
            full((HID, HID)), full((1, HID)),
            full((HID, PAD)), full((1, PAD)),
            pl.BlockSpec((1, BM, 1), lambda i: (i, 0, 0)),
            pl.BlockSpec((1, BM, 1), lambda i: (i, 0, 0)),
        ],
        out_specs=[
            pl.BlockSpec((1, 1), lambda i: (0, 0), memory_space=pltpu.SMEM),
            pl.BlockSpec((1, 1), lambda i: (0, 0), memory_space=pltpu.SMEM),
            pl.BlockSpec((1, 1), lambda i: (0, 0), memory_space=pltpu.SMEM),
        ],
    )
    sums = pl.pallas_call(
        _losses_kernel,
        grid_spec=grid_spec,
        out_shape=[jax.ShapeDtypeStruct((1, 1), jnp.float32)] * 3,
    )(
        seq_repr,
        dW1, db1.reshape(1, HID), dW2, db2.reshape(1, HID), dW3p, db3p,
        cW1, cb1.reshape(1, HID), cW2, cb2.reshape(1, HID), cW3p, cb3p,
        dlab, clab,
    )
    deno_sum, cono_sum, adv_sum = sums
    deno_loss = -deno_sum[0, 0] / B
    cono_loss = -cono_sum[0, 0] / B
    adv_loss = adv_sum[0, 0] / B
    return deno_loss, cono_loss, adv_loss


def kernel(table, dW1, db1, dW2, db2, dW3, db3, cW1, cb1, cW2, cb2, cW3, cb3,
           seq_word_ids, deno_labels, cono_labels):
    ids = seq_word_ids.astype(jnp.int32).reshape(B * L // SUB_ROWS, SUB_ROWS)
    vecs_flat, seq_repr = _gather_mean(ids, table)
    seq_word_vecs = vecs_flat.reshape(B, L, DIM)

    deno_loss, cono_loss, adv_loss = _run_losses(
        seq_repr, dW1, db1, dW2, db2, dW3, db3, cW1, cb1, cW2, cb2, cW3, cb3,
        deno_labels, cono_labels,
    )
    return (deno_loss, cono_loss, adv_loss, seq_word_vecs)


# speedup vs baseline: 4.1198x; 4.1198x over previous
import functools

import jax
import jax.numpy as jnp
from jax import lax
from jax.experimental import pallas as pl
from jax.experimental.pallas import tpu as pltpu
from jax.experimental.pallas import tpu_sc as plsc

VOCAB = 100000
DIM = 128
HID = 1024
N_DENO = 41
N_CONO = 2
B = 16384
L = 50

NC = 2
NS = 16
NW = NC * NS

SAMP_PER_W = B // NW
CHUNK_SAMP = 8
ROWS_PER_CHUNK = CHUNK_SAMP * L
SUB = 5
SUB_ROWS = ROWS_PER_CHUNK // SUB
N_CHUNK = SAMP_PER_W // CHUNK_SAMP

_sc_mesh = plsc.VectorSubcoreMesh(core_axis_name="c", subcore_axis_name="s")


@functools.partial(
    pl.kernel,
    mesh=_sc_mesh,
    out_type=[
        jax.ShapeDtypeStruct((B * L, DIM), jnp.float32),
        jax.ShapeDtypeStruct((B, DIM), jnp.float32),
    ],
    scratch_types=[
        pltpu.VMEM((ROWS_PER_CHUNK,), jnp.int32),
        pltpu.VMEM((ROWS_PER_CHUNK,), jnp.int32),
        pltpu.VMEM((ROWS_PER_CHUNK, DIM), jnp.float32),
        pltpu.VMEM((ROWS_PER_CHUNK, DIM), jnp.float32),
        pltpu.VMEM((CHUNK_SAMP, DIM), jnp.float32),
        pltpu.VMEM((CHUNK_SAMP, DIM), jnp.float32),
        pltpu.SemaphoreType.DMA,
        pltpu.SemaphoreType.DMA,
        pltpu.SemaphoreType.DMA,
        pltpu.SemaphoreType.DMA,
        pltpu.SemaphoreType.DMA,
        pltpu.SemaphoreType.DMA,
    ],
)
def _gather_mean(ids_hbm, table_hbm, vecs_hbm, repr_hbm,
                 idx_a, idx_b, rows_a, rows_b, repr_a, repr_b,
                 si_a, si_b, sg_a, sg_b, sw_a, sw_b):
    wid = lax.axis_index("s") * NC + lax.axis_index("c")
    base_samp = wid * SAMP_PER_W

    def row0_of(g):
        return (base_samp + g * CHUNK_SAMP) * L

    def samp0_of(g):
        return base_samp + g * CHUNK_SAMP

    def start_idx(g, idx, sem):
        pltpu.async_copy(ids_hbm.at[pl.ds(row0_of(g), ROWS_PER_CHUNK)], idx, sem)

    def wait_idx(idx, sem):
        pltpu.make_async_copy(
            ids_hbm.at[pl.ds(0, ROWS_PER_CHUNK)], idx, sem).wait()

    def start_gathers(idx, rows, sem):
        for k in range(SUB):
            pltpu.async_copy(
                table_hbm.at[idx.at[pl.ds(k * SUB_ROWS, SUB_ROWS)]],
                rows.at[pl.ds(k * SUB_ROWS, SUB_ROWS), :],
                sem,
            )

    def wait_gathers(idx, rows, sem):
        for k in range(SUB):
            pltpu.make_async_copy(
                table_hbm.at[idx.at[pl.ds(k * SUB_ROWS, SUB_ROWS)]],
                rows.at[pl.ds(k * SUB_ROWS, SUB_ROWS), :],
                sem,
            ).wait()

    def start_writes(g, rows, repr_v, sem):
        pltpu.async_copy(rows, vecs_hbm.at[pl.ds(row0_of(g), ROWS_PER_CHUNK), :], sem)
        pltpu.async_copy(repr_v, repr_hbm.at[pl.ds(samp0_of(g), CHUNK_SAMP), :], sem)

    def wait_writes(rows, repr_v, sem):
        pltpu.make_async_copy(
            rows, vecs_hbm.at[pl.ds(0, ROWS_PER_CHUNK), :], sem).wait()
        pltpu.make_async_copy(
            repr_v, repr_hbm.at[pl.ds(0, CHUNK_SAMP), :], sem).wait()

    def accumulate(rows, repr_v):
        for j in range(CHUNK_SAMP):
            def rbody(it, acc):
                for rr in range(5):
                    r = it * 5 + rr
                    acc = tuple(
                        acc[c] + rows[j * L + r, pl.ds(c * 16, 16)]
                        for c in range(8)
                    )
                return acc
            acc0 = tuple(jnp.zeros((16,), jnp.float32) for _ in range(8))
            acc = lax.fori_loop(0, L // 5, rbody, acc0)
            for c in range(8):
                repr_v[j, pl.ds(c * 16, 16)] = acc[c] * (1.0 / L)

    bufs = (
        (idx_a, rows_a, repr_a, si_a, sg_a, sw_a),
        (idx_b, rows_b, repr_b, si_b, sg_b, sw_b),
    )

    start_idx(0, idx_a, si_a)
    start_idx(1, idx_b, si_b)
    wait_idx(idx_a, si_a)
    start_gathers(idx_a, rows_a, sg_a)

    def pair_body(t, carry):
        for off in range(2):
            idx_c, rows_c, repr_c, si_c, sg_c, sw_c = bufs[off]
            idx_n, rows_n, repr_n, si_n, sg_n, sw_n = bufs[1 - off]
            g = 2 * t + off
            wait_gathers(idx_c, rows_c, sg_c)

            @pl.when(g > 0)
            def _():
                wait_writes(rows_n, repr_n, sw_n)

            @pl.when(g + 1 < N_CHUNK)
            def _():
                wait_idx(idx_n, si_n)
                start_gathers(idx_n, rows_n, sg_n)

            @pl.when(g + 2 < N_CHUNK)
            def _():
                start_idx(g + 2, idx_c, si_c)

            accumulate(rows_c, repr_c)
            start_writes(g, rows_c, repr_c, sw_c)
        return carry

    lax.fori_loop(0, N_CHUNK // 2, pair_body, 0)
    wait_writes(rows_b, repr_b, sw_b)


BM = 1024
NB = B // BM
PAD = 128


def _probe_block(x, W1, b1, W2, b2, W3, b3):
    h = jnp.maximum(jnp.dot(x, W1, preferred_element_type=jnp.float32) + b1, 0.0)
    h = jnp.maximum(jnp.dot(h, W2, preferred_element_type=jnp.float32) + b2, 0.0)
    return jnp.dot(h, W3, preferred_element_type=jnp.float32) + b3


def _log_softmax(logits):
    m = jnp.max(logits, axis=1, keepdims=True)
    s = logits - m
    lse = jnp.log(jnp.sum(jnp.exp(s), axis=1, keepdims=True))
    return s - lse


def _losses_kernel(
    repr_ref, dW1, db1, dW2, db2, dW3, db3, cW1, cb1, cW2, cb2, cW3, cb3,
    dlab_ref, clab_ref, deno_sum, cono_sum, adv_sum,
):
    i = pl.program_id(0)

    @pl.when(i == 0)
    def _():
        deno_sum[0, 0] = 0.0
        cono_sum[0, 0] = 0.0
        adv_sum[0, 0] = 0.0

    x = repr_ref[...]

    cols = lax.broadcasted_iota(jnp.int32, (BM, PAD), 1)

    dlogits = _probe_block(x, dW1[...], db1[...], dW2[...], db2[...], dW3[...], db3[...])
    dlogp = _log_softmax(dlogits)
    d_onehot = (cols == dlab_ref[0]).astype(jnp.float32)
    deno_sum[0, 0] += jnp.sum(dlogp * d_onehot)

    clogits = _probe_block(x, cW1[...], cb1[...], cW2[...], cb2[...], cW3[...], cb3[...])
    clogp = _log_softmax(clogits)
    c_onehot = (cols == clab_ref[0]).astype(jnp.float32)
    cono_sum[0, 0] += jnp.sum(clogp * c_onehot)

    u = 1.0 / N_CONO
    kl_terms = jnp.where(cols < N_CONO, u * (jnp.log(u) - clogp), 0.0)
    adv_sum[0, 0] += jnp.sum(kl_terms)


def _run_losses(seq_repr, dW1, db1, dW2, db2, dW3, db3, cW1, cb1, cW2, cb2, cW3, cb3,
                deno_labels, cono_labels):
    dW3p = jnp.pad(dW3, ((0, 0), (0, PAD - N_DENO)))
    db3p = jnp.pad(db3.reshape(1, N_DENO), ((0, 0), (0, PAD - N_DENO)),
                   constant_values=-1e30)
    cW3p = jnp.pad(cW3, ((0, 0), (0, PAD - N_CONO)))
    cb3p = jnp.pad(cb3.reshape(1, N_CONO), ((0, 0), (0, PAD - N_CONO)),
                   constant_values=-1e30)

    dlab = deno_labels.astype(jnp.int32).reshape(NB, BM, 1)
    clab = cono_labels.astype(jnp.int32).reshape(NB, BM, 1)

    full = lambda shape: pl.BlockSpec(shape, lambda i: (0,) * len(shape))
    grid_spec = pl.GridSpec(
        grid=(NB,),
        in_specs=[
            pl.BlockSpec((BM, DIM), lambda i: (i, 0)),
            full((DIM, HID)), full((1, HID)),
            full((HID, HID)), full((1, HID)),
            full((HID, PAD)), full((1, PAD)),
            full((DIM, HID)), full((1, HID)),
            full((HID, HID)), full((1, HID)),
            full((HID, PAD)), full((1, PAD)),
            pl.BlockSpec((1, BM, 1), lambda i: (i, 0, 0)),
            pl.BlockSpec((1, BM, 1), lambda i: (i, 0, 0)),
        ],
        out_specs=[
            pl.BlockSpec((1, 1), lambda i: (0, 0), memory_space=pltpu.SMEM),
            pl.BlockSpec((1, 1), lambda i: (0, 0), memory_space=pltpu.SMEM),
            pl.BlockSpec((1, 1), lambda i: (0, 0), memory_space=pltpu.SMEM),
        ],
    )
    sums = pl.pallas_call(
        _losses_kernel,
        grid_spec=grid_spec,
        out_shape=[jax.ShapeDtypeStruct((1, 1), jnp.float32)] * 3,
    )(
        seq_repr,
        dW1, db1.reshape(1, HID), dW2, db2.reshape(1, HID), dW3p, db3p,
        cW1, cb1.reshape(1, HID), cW2, cb2.reshape(1, HID), cW3p, cb3p,
        dlab, clab,
    )
    deno_sum, cono_sum, adv_sum = sums
    deno_loss = -deno_sum[0, 0] / B
    cono_loss = -cono_sum[0, 0] / B
    adv_loss = adv_sum[0, 0] / B
    return deno_loss, cono_loss, adv_loss


def kernel(table, dW1, db1, dW2, db2, dW3, db3, cW1, cb1, cW2, cb2, cW3, cb3,
           seq_word_ids, deno_labels, cono_labels):
    ids = seq_word_ids.astype(jnp.int32).reshape(B * L)
    vecs_flat, seq_repr = _gather_mean(ids, table)
    seq_word_vecs = vecs_flat.reshape(B, L, DIM)

    deno_loss, cono_loss, adv_loss = _run_losses(
        seq_repr, dW1, db1, dW2, db2, dW3, db3, cW1, cb1, cW2, cb2, cW3, cb3,
        deno_labels, cono_labels,
    )
    return (deno_loss, cono_loss, adv_loss, seq_word_vecs)
